# baseline (speedup 1.0000x reference)
import functools

import jax
import jax.numpy as jnp
from jax import lax
from jax.experimental import pallas as pl
from jax.experimental.pallas import tpu as pltpu

N_TOK = 4096
HIDDEN = 2048
FF = 1024
N_ROUTED = 8
TOTAL_EXPERTS = 10
TOP_K = 2
SCALE = 2.5

TILE = 256
A = N_TOK * TOP_K
NT = (A + N_ROUTED * (TILE - 1) + TILE - 1) // TILE
P = NT * TILE
RB = 512


def _router_kernel(h_ref, wc_ref, bias_ref, sel_ref, w_ref, idwh_ref):
    h = h_ref[...]
    logits = lax.dot_general(h, wc_ref[...], (((1,), (1,)), ((), ())),
                             preferred_element_type=jnp.float32)
    m = jnp.max(logits, axis=-1, keepdims=True)
    e = jnp.exp(logits - m)
    probs = e / jnp.sum(e, axis=-1, keepdims=True)
    biased = probs + bias_ref[...]
    iota = lax.broadcasted_iota(jnp.int32, biased.shape, 1)
    big = jnp.full_like(iota, TOTAL_EXPERTS)
    v1 = jnp.max(biased, axis=-1, keepdims=True)
    i1 = jnp.min(jnp.where(biased == v1, iota, big), axis=-1, keepdims=True)
    b2 = jnp.where(iota == i1, -jnp.inf, biased)
    v2 = jnp.max(b2, axis=-1, keepdims=True)
    i2 = jnp.min(jnp.where(b2 == v2, iota, big), axis=-1, keepdims=True)
    w1 = jnp.sum(jnp.where(iota == i1, probs, 0.0), axis=-1, keepdims=True) * SCALE
    w2 = jnp.sum(jnp.where(iota == i2, probs, 0.0), axis=-1, keepdims=True) * SCALE
    sel_ref[...] = jnp.concatenate([i1, i2], axis=1)
    w_ref[...] = jnp.concatenate([w1, w2], axis=1)
    id_w = w1 * (i1 >= N_ROUTED) + w2 * (i2 >= N_ROUTED)
    idwh_ref[...] = h * id_w


def _router(h, wc, bias, interpret=False):
    grid = (N_TOK // RB,)
    return pl.pallas_call(
        _router_kernel,
        grid=grid,
        in_specs=[
            pl.BlockSpec((RB, HIDDEN), lambda i: (i, 0)),
            pl.BlockSpec((TOTAL_EXPERTS, HIDDEN), lambda i: (0, 0)),
            pl.BlockSpec((1, TOTAL_EXPERTS), lambda i: (0, 0)),
        ],
        out_specs=[
            pl.BlockSpec((RB, TOP_K), lambda i: (i, 0)),
            pl.BlockSpec((RB, TOP_K), lambda i: (i, 0)),
            pl.BlockSpec((RB, HIDDEN), lambda i: (i, 0)),
        ],
        out_shape=[
            jax.ShapeDtypeStruct((N_TOK, TOP_K), jnp.int32),
            jax.ShapeDtypeStruct((N_TOK, TOP_K), jnp.float32),
            jax.ShapeDtypeStruct((N_TOK, HIDDEN), jnp.float32),
        ],
        interpret=interpret,
    )(h, wc, bias.reshape(1, TOTAL_EXPERTS))


def _mlp_kernel(te_ref, hg_ref, gw_ref, uw_ref, dw_ref, rw_ref, out_ref):
    hg = hg_ref[...].astype(jnp.bfloat16)
    g = lax.dot_general(hg, gw_ref[0], (((1,), (1,)), ((), ())),
                        preferred_element_type=jnp.float32)
    u = lax.dot_general(hg, uw_ref[0], (((1,), (1,)), ((), ())),
                        preferred_element_type=jnp.float32)
    inter = (g * jax.nn.sigmoid(g)) * u * rw_ref[0]
    out_ref[...] = lax.dot_general(inter.astype(jnp.bfloat16), dw_ref[0],
                                   (((1,), (1,)), ((), ())),
                                   preferred_element_type=jnp.float32)


def _grouped_mlp(hg, gw, uw, dw, row_w, tile_expert, interpret=False):
    rw3 = row_w.reshape(NT, TILE, 1)
    grid_spec = pltpu.PrefetchScalarGridSpec(
        num_scalar_prefetch=1,
        grid=(NT,),
        in_specs=[
            pl.BlockSpec((TILE, HIDDEN), lambda i, te: (i, 0)),
            pl.BlockSpec((1, FF, HIDDEN), lambda i, te: (te[i], 0, 0)),
            pl.BlockSpec((1, FF, HIDDEN), lambda i, te: (te[i], 0, 0)),
            pl.BlockSpec((1, HIDDEN, FF), lambda i, te: (te[i], 0, 0)),
            pl.BlockSpec((1, TILE, 1), lambda i, te: (i, 0, 0)),
        ],
        out_specs=pl.BlockSpec((TILE, HIDDEN), lambda i, te: (i, 0)),
    )
    return pl.pallas_call(
        _mlp_kernel,
        grid_spec=grid_spec,
        out_shape=jax.ShapeDtypeStruct((P, HIDDEN), jnp.float32),
        compiler_params=pltpu.CompilerParams(
            dimension_semantics=("arbitrary",)),
        interpret=interpret,
    )(tile_expert, hg, gw, uw, dw, rw3)


def _combine_kernel(idwh_ref, g0_ref, g1_ref, out_ref):
    out_ref[...] = idwh_ref[...] + g0_ref[...] + g1_ref[...]


def _combine(idwh, g0, g1, interpret=False):
    grid = (N_TOK // RB,)
    bs = pl.BlockSpec((RB, HIDDEN), lambda i: (i, 0))
    return pl.pallas_call(
        _combine_kernel,
        grid=grid,
        in_specs=[bs, bs, bs],
        out_specs=bs,
        out_shape=jax.ShapeDtypeStruct((N_TOK, HIDDEN), jnp.float32),
        interpret=interpret,
    )(idwh, g0, g1)


def _dispatch_indices(sel, w):
    eid = sel.reshape(-1)
    wf = w.reshape(-1)
    routed = eid < N_ROUTED
    onehot = (eid[:, None] == jnp.arange(N_ROUTED)[None, :]).astype(jnp.int32)
    ranks_all = jnp.cumsum(onehot, axis=0) - onehot
    rank = jnp.sum(ranks_all * onehot, axis=1)
    counts = jnp.sum(onehot, axis=0)
    padded = ((counts + TILE - 1) // TILE) * TILE
    starts = jnp.concatenate([jnp.zeros(1, padded.dtype), jnp.cumsum(padded)[:-1]])
    pos = jnp.where(routed, starts[jnp.clip(eid, 0, N_ROUTED - 1)] + rank,
                    P - 1).astype(jnp.int32)
    a_token = (jnp.arange(A) // TOP_K).astype(jnp.int32)
    token_of_pos = jnp.zeros((P,), jnp.int32).at[pos].set(
        jnp.where(routed, a_token, 0))
    row_w = jnp.zeros((P,), jnp.float32).at[pos].set(jnp.where(routed, wf, 0.0))
    ends = starts + padded
    tile_expert = jnp.minimum(
        jnp.sum((jnp.arange(NT)[:, None] * TILE >= ends[None, :]).astype(jnp.int32),
                axis=1), N_ROUTED - 1).astype(jnp.int32)
    return pos, token_of_pos, row_w, tile_expert


def kernel(hidden_states, classifier_w, e_score_correction_bias,
           gate_w, up_w, down_w, interpret=False):
    sel, w, idwh = _router(hidden_states, classifier_w,
                           e_score_correction_bias, interpret=interpret)
    pos, token_of_pos, row_w, tile_expert = _dispatch_indices(sel, w)
    hg = jnp.take(hidden_states, token_of_pos, axis=0)
    ogw = _grouped_mlp(hg, gate_w.astype(jnp.bfloat16),
                       up_w.astype(jnp.bfloat16), down_w.astype(jnp.bfloat16),
                       row_w, tile_expert, interpret=interpret)
    p01 = pos.reshape(N_TOK, TOP_K)
    g0 = jnp.take(ogw, p01[:, 0], axis=0)
    g1 = jnp.take(ogw, p01[:, 1], axis=0)
    return _combine(idwh, g0, g1, interpret=interpret)
